# baseline (speedup 1.0000x reference)
import jax
import jax.numpy as jnp
from jax.experimental import pallas as pl
from jax.experimental.pallas import tpu as pltpu

_N = 10000
_D = 128
_ALPHA = 3.0
_K = 16
_BR = 80
_CT = 1280
_NFULL = 7
_TILES = [(t * _CT, _CT) for t in range(_NFULL)] + [(_NFULL * _CT, _N - _NFULL * _CT)]


def _prep_kernel(e_ref, r_ref, m1_ref, m2_ref):
    m1_ref[...] = jnp.tanh(_ALPHA * e_ref[...]).astype(jnp.bfloat16)
    m2_ref[...] = jnp.tanh(_ALPHA * r_ref[...]).astype(jnp.bfloat16)


def _dot_nt(a, b):
    return jax.lax.dot_general(
        a, b, (((1,), (1,)), ((), ())),
        preferred_element_type=jnp.float32,
    )


def _main_kernel(m1b_ref, m2b_ref, m1_ref, m2_ref, out_ref, aw_ref):
    m1b = m1b_ref[...]
    m2b = m2b_ref[...]

    for off, w in _TILES:
        m1t = m1_ref[pl.ds(off, w), :]
        m2t = m2_ref[pl.ds(off, w), :]
        s = _dot_nt(m1b, m2t) - _dot_nt(m2b, m1t)
        aw_ref[:, pl.ds(off, w)] = jnp.maximum(jnp.tanh(_ALPHA * s), 0.0)

    def cond(carry):
        _, cum, _ = carry
        return jnp.any(cum < _K)

    def body(carry):
        t, cum, c_last = carry
        best_v = jnp.full((_BR, 1), -1.0, dtype=jnp.float32)
        cnt = jnp.zeros((_BR, 1), dtype=jnp.int32)
        for off, w in _TILES:
            a = aw_ref[:, pl.ds(off, w)]
            v = jnp.where(a < t, a, -1.0)
            mt = jnp.max(v, axis=1, keepdims=True)
            ct = jnp.sum((v == mt).astype(jnp.int32), axis=1, keepdims=True)
            valid = mt >= 0.0
            gt = valid & (mt > best_v)
            eq = valid & (mt == best_v)
            cnt = jnp.where(gt, ct, jnp.where(eq, cnt + ct, cnt))
            best_v = jnp.where(gt, mt, best_v)
        active = cum < _K
        t = jnp.where(active, best_v, t)
        cum = jnp.where(active, cum + cnt, cum)
        c_last = jnp.where(active, cnt, c_last)
        return t, cum, c_last

    t0 = jnp.full((_BR, 1), jnp.inf, dtype=jnp.float32)
    z0 = jnp.zeros((_BR, 1), dtype=jnp.int32)
    t, cum, c_last = jax.lax.while_loop(cond, body, (t0, z0, z0))
    r = _K - (cum - c_last)

    carry = jnp.zeros((_BR, 1), dtype=jnp.int32)
    it128 = jax.lax.broadcasted_iota(jnp.int32, (_BR, 128), 1)
    for off in range(0, _N, 128):
        w = min(128, _N - off)
        a = aw_ref[:, pl.ds(off, w)]
        eq = a == t
        eqi = eq.astype(jnp.int32)
        itw = it128 if w == 128 else it128[:, :w]
        x = eqi
        s = 1
        while s < w:
            x = x + jnp.where(itw >= s, pltpu.roll(x, s, 1), 0)
            s *= 2
        pc = carry + x - eqi
        keep = (a > t) | (eq & (pc < r))
        out_ref[:, pl.ds(off, w)] = jnp.where(keep, a, 0.0)
        carry = carry + x[:, w - 1:w]


def kernel(emb_emitter, emb_receiver):
    m1, m2 = pl.pallas_call(
        _prep_kernel,
        out_shape=[jax.ShapeDtypeStruct((_N, _D), jnp.bfloat16)] * 2,
    )(emb_emitter, emb_receiver)

    out = pl.pallas_call(
        _main_kernel,
        grid=(_N // _BR,),
        in_specs=[
            pl.BlockSpec((_BR, _D), lambda i: (i, 0)),
            pl.BlockSpec((_BR, _D), lambda i: (i, 0)),
            pl.BlockSpec((_N, _D), lambda i: (0, 0)),
            pl.BlockSpec((_N, _D), lambda i: (0, 0)),
        ],
        out_specs=pl.BlockSpec((_BR, _N), lambda i: (i, 0)),
        out_shape=jax.ShapeDtypeStruct((_N, _N), jnp.float32),
        scratch_shapes=[pltpu.VMEM((_BR, _N), jnp.float32)],
        compiler_params=pltpu.CompilerParams(
            dimension_semantics=("parallel",),
        ),
    )(m1, m2, m1, m2)
    return out


# speedup vs baseline: 6.1649x; 6.1649x over previous
import jax
import jax.numpy as jnp
from jax.experimental import pallas as pl
from jax.experimental.pallas import tpu as pltpu

_N = 10000
_D = 128
_ALPHA = 3.0
_K = 16
_BR = 80
_CT = 1280
_NFULL = 7
_TILES = [(t * _CT, _CT) for t in range(_NFULL)] + [(_NFULL * _CT, _N - _NFULL * _CT)]


def _prep_kernel(e_ref, r_ref, m1_ref, m2_ref):
    m1_ref[...] = jnp.tanh(_ALPHA * e_ref[...]).astype(jnp.bfloat16)
    m2_ref[...] = jnp.tanh(_ALPHA * r_ref[...]).astype(jnp.bfloat16)


def _dot_nt(a, b):
    return jax.lax.dot_general(
        a, b, (((1,), (1,)), ((), ())),
        preferred_element_type=jnp.float32,
    )


def _main_kernel(m1b_ref, m2b_ref, m1_ref, m2_ref, out_ref, aw_ref):
    m1b = m1b_ref[...]
    m2b = m2b_ref[...]

    for off, w in _TILES:
        m1t = m1_ref[pl.ds(off, w), :]
        m2t = m2_ref[pl.ds(off, w), :]
        s = _dot_nt(m1b, m2t) - _dot_nt(m2b, m1t)
        aw_ref[:, pl.ds(off, w)] = jnp.maximum(jnp.tanh(_ALPHA * s), 0.0)

    def cond(carry):
        _, cum, _ = carry
        return jnp.any(cum < _K)

    def body(carry):
        t, cum, c_last = carry
        best_v = jnp.full((_BR, 1), -1.0, dtype=jnp.float32)
        cnt = jnp.zeros((_BR, 1), dtype=jnp.int32)
        for off, w in _TILES:
            a = aw_ref[:, pl.ds(off, w)]
            v = jnp.where(a < t, a, -1.0)
            mt = jnp.max(v, axis=1, keepdims=True)
            ct = jnp.sum((v == mt).astype(jnp.int32), axis=1, keepdims=True)
            valid = mt >= 0.0
            gt = valid & (mt > best_v)
            eq = valid & (mt == best_v)
            cnt = jnp.where(gt, ct, jnp.where(eq, cnt + ct, cnt))
            best_v = jnp.where(gt, mt, best_v)
        active = cum < _K
        t = jnp.where(active, best_v, t)
        cum = jnp.where(active, cum + cnt, cum)
        c_last = jnp.where(active, cnt, c_last)
        return t, cum, c_last

    t0 = jnp.full((_BR, 1), jnp.inf, dtype=jnp.float32)
    z0 = jnp.zeros((_BR, 1), dtype=jnp.int32)
    t, cum, c_last = jax.lax.while_loop(cond, body, (t0, z0, z0))
    r = _K - (cum - c_last)

    cts = []
    for off, w in _TILES:
        a = aw_ref[:, pl.ds(off, w)]
        cts.append(jnp.sum((a == t).astype(jnp.int32), axis=1, keepdims=True))
    cum = jnp.zeros((_BR, 1), dtype=jnp.int32)
    tau = jnp.full((_BR, 1), -1, dtype=jnp.int32)
    rbef = jnp.zeros((_BR, 1), dtype=jnp.int32)
    for ti in range(len(_TILES)):
        ncum = cum + cts[ti]
        hit = (tau < 0) & (ncum >= r)
        tau = jnp.where(hit, ti, tau)
        rbef = jnp.where(hit, cum, rbef)
        cum = ncum
    acc = jnp.zeros((_BR, _CT), dtype=jnp.int32)
    for ti, (off, w) in enumerate(_TILES):
        a = aw_ref[:, pl.ds(off, w)]
        eq = jnp.where(a == t, 1, 0)
        if w < _CT:
            eq = jnp.concatenate(
                [eq, jnp.zeros((_BR, _CT - w), dtype=jnp.int32)], axis=1)
        acc = jnp.where(tau == ti, eq, acc)
    itc = jax.lax.broadcasted_iota(jnp.int32, (_BR, _CT), 1)
    x = acc
    s = 1
    while s < _CT:
        x = x + jnp.where(itc >= s, pltpu.roll(x, s, 1), 0)
        s *= 2
    r2 = r - rbef
    fl = jnp.min(jnp.where((x == r2) & (acc == 1), itc, _CT), axis=1,
                 keepdims=True)
    f = tau * _CT + fl

    for off, w in _TILES:
        a = aw_ref[:, pl.ds(off, w)]
        it = jax.lax.broadcasted_iota(jnp.int32, a.shape, 1) + off
        keep = (a > t) | ((a == t) & (it <= f))
        out_ref[:, pl.ds(off, w)] = jnp.where(keep, a, 0.0)


def kernel(emb_emitter, emb_receiver):
    m1, m2 = pl.pallas_call(
        _prep_kernel,
        out_shape=[jax.ShapeDtypeStruct((_N, _D), jnp.bfloat16)] * 2,
    )(emb_emitter, emb_receiver)

    out = pl.pallas_call(
        _main_kernel,
        grid=(_N // _BR,),
        in_specs=[
            pl.BlockSpec((_BR, _D), lambda i: (i, 0)),
            pl.BlockSpec((_BR, _D), lambda i: (i, 0)),
            pl.BlockSpec((_N, _D), lambda i: (0, 0)),
            pl.BlockSpec((_N, _D), lambda i: (0, 0)),
        ],
        out_specs=pl.BlockSpec((_BR, _N), lambda i: (i, 0)),
        out_shape=jax.ShapeDtypeStruct((_N, _N), jnp.float32),
        scratch_shapes=[pltpu.VMEM((_BR, _N), jnp.float32)],
        compiler_params=pltpu.CompilerParams(
            dimension_semantics=("parallel",),
        ),
    )(m1, m2, m1, m2)
    return out


# speedup vs baseline: 7.3125x; 1.1862x over previous
import jax
import jax.numpy as jnp
from jax.experimental import pallas as pl
from jax.experimental.pallas import tpu as pltpu

_N = 10000
_NP = 10240
_D = 128
_ALPHA = 3.0
_K = 16
_BR = 80
_CT = 1280
_NT = _NP // _CT
_TAILW = _N - (_NT - 1) * _CT


def _prep_kernel(e_ref, r_ref, m1_ref, m2_ref):
    m1_ref[pl.ds(0, _N), :] = jnp.tanh(_ALPHA * e_ref[...]).astype(jnp.bfloat16)
    m2_ref[pl.ds(0, _N), :] = jnp.tanh(_ALPHA * r_ref[...]).astype(jnp.bfloat16)
    pad = jnp.zeros((_NP - _N, _D), dtype=jnp.bfloat16)
    m1_ref[pl.ds(_N, _NP - _N), :] = pad
    m2_ref[pl.ds(_N, _NP - _N), :] = pad


def _dot_nt(a, b):
    return jax.lax.dot_general(
        a, b, (((1,), (1,)), ((), ())),
        preferred_element_type=jnp.float32,
    )


def _main_kernel(m1b_ref, m2b_ref, m1_ref, m2_ref, out_ref, aw_ref):
    m1b = m1b_ref[...]
    m2b = m2b_ref[...]
    itc = jax.lax.broadcasted_iota(jnp.int32, (_BR, _CT), 1)
    it128 = jax.lax.broadcasted_iota(jnp.int32, (_BR, 128), 1)

    best_v = jnp.full((_BR, 1), -1.0, dtype=jnp.float32)
    cnt = jnp.zeros((_BR, 1), dtype=jnp.int32)
    for ti in range(_NT):
        off = ti * _CT
        m1t = m1_ref[pl.ds(off, _CT), :]
        m2t = m2_ref[pl.ds(off, _CT), :]
        s = _dot_nt(m1b, m2t) - _dot_nt(m2b, m1t)
        a = jnp.maximum(jnp.tanh(_ALPHA * s), 0.0)
        if ti == _NT - 1:
            a = jnp.where(itc >= _TAILW, -1.0, a)
        aw_ref[:, pl.ds(off, _CT)] = a
        mt = jnp.max(a, axis=1, keepdims=True)
        ct = jnp.sum(jnp.where(a == mt, 1, 0), axis=1, keepdims=True)
        gt = mt > best_v
        eq = mt == best_v
        cnt = jnp.where(gt, ct, jnp.where(eq, cnt + ct, cnt))
        best_v = jnp.where(gt, mt, best_v)

    def cond(carry):
        _, cum, _ = carry
        return jnp.any(cum < _K)

    def body(carry):
        t, cum, c_last = carry
        bv = jnp.full((_BR, 1), -1.0, dtype=jnp.float32)
        cn = jnp.zeros((_BR, 1), dtype=jnp.int32)
        for ti in range(_NT):
            a = aw_ref[:, pl.ds(ti * _CT, _CT)]
            v = jnp.where(a < t, a, -1.0)
            mt = jnp.max(v, axis=1, keepdims=True)
            ct = jnp.sum(jnp.where(v == mt, 1, 0), axis=1, keepdims=True)
            valid = mt >= 0.0
            gt = valid & (mt > bv)
            eq = valid & (mt == bv)
            cn = jnp.where(gt, ct, jnp.where(eq, cn + ct, cn))
            bv = jnp.where(gt, mt, bv)
        active = cum < _K
        t = jnp.where(active, bv, t)
        cum = jnp.where(active, cum + cn, cum)
        c_last = jnp.where(active, cn, c_last)
        return t, cum, c_last

    t, cum, c_last = jax.lax.while_loop(cond, body, (best_v, cnt, cnt))
    r = _K - (cum - c_last)

    cum2 = jnp.zeros((_BR, 1), dtype=jnp.int32)
    tau = jnp.full((_BR, 1), -1, dtype=jnp.int32)
    rbef = jnp.zeros((_BR, 1), dtype=jnp.int32)
    acc = jnp.zeros((_BR, _CT), dtype=jnp.int32)
    for ti in range(_NT):
        a = aw_ref[:, pl.ds(ti * _CT, _CT)]
        eq = jnp.where(a == t, 1, 0)
        ct = jnp.sum(eq, axis=1, keepdims=True)
        ncum = cum2 + ct
        hit = (tau < 0) & (ncum >= r)
        acc = jnp.where(hit, eq, acc)
        tau = jnp.where(hit, ti, tau)
        rbef = jnp.where(hit, cum2, rbef)
        cum2 = ncum
    cumc = jnp.zeros((_BR, 1), dtype=jnp.int32)
    cpos = jnp.full((_BR, 1), -1, dtype=jnp.int32)
    rbefc = jnp.zeros((_BR, 1), dtype=jnp.int32)
    acc2 = jnp.zeros((_BR, 128), dtype=jnp.int32)
    r2 = r - rbef
    for ci in range(_CT // 128):
        ch = acc[:, ci * 128:(ci + 1) * 128]
        cc = jnp.sum(ch, axis=1, keepdims=True)
        ncum = cumc + cc
        hit = (cpos < 0) & (ncum >= r2)
        acc2 = jnp.where(hit, ch, acc2)
        cpos = jnp.where(hit, ci, cpos)
        rbefc = jnp.where(hit, cumc, rbefc)
        cumc = ncum
    r3 = r2 - rbefc
    x = acc2
    s = 1
    while s < 128:
        x = x + jnp.where(it128 >= s, pltpu.roll(x, s, 1), 0)
        s *= 2
    fl = jnp.min(jnp.where((x == r3) & (acc2 == 1), it128, 128), axis=1,
                 keepdims=True)
    f = tau * _CT + cpos * 128 + fl

    for ti in range(_NT):
        off = ti * _CT
        w = _CT if ti < _NT - 1 else _TAILW
        a = aw_ref[:, pl.ds(off, w)]
        it = jax.lax.broadcasted_iota(jnp.int32, a.shape, 1) + off
        keep = (a > t) | ((a == t) & (it <= f))
        out_ref[:, pl.ds(off, w)] = jnp.where(keep, a, 0.0)


def kernel(emb_emitter, emb_receiver):
    m1, m2 = pl.pallas_call(
        _prep_kernel,
        out_shape=[jax.ShapeDtypeStruct((_NP, _D), jnp.bfloat16)] * 2,
    )(emb_emitter, emb_receiver)

    out = pl.pallas_call(
        _main_kernel,
        grid=(_N // _BR,),
        in_specs=[
            pl.BlockSpec((_BR, _D), lambda i: (i, 0)),
            pl.BlockSpec((_BR, _D), lambda i: (i, 0)),
            pl.BlockSpec((_NP, _D), lambda i: (0, 0)),
            pl.BlockSpec((_NP, _D), lambda i: (0, 0)),
        ],
        out_specs=pl.BlockSpec((_BR, _N), lambda i: (i, 0)),
        out_shape=jax.ShapeDtypeStruct((_N, _N), jnp.float32),
        scratch_shapes=[pltpu.VMEM((_BR, _NP), jnp.float32)],
        compiler_params=pltpu.CompilerParams(
            dimension_semantics=("parallel",),
        ),
    )(m1, m2, m1, m2)
    return out
